# baseline (speedup 1.0000x reference)
import functools

import jax
import jax.numpy as jnp
from jax import lax
from jax.experimental import pallas as pl
from jax.experimental.pallas import tpu as pltpu
import jax.experimental.pallas.tpu_sc as plsc

N_NODES = 10000
N_PAD = 10240
D_FEAT = 128
HIDDEN = 128
N_CLASSES = 40
C_PAD = 48

N_SUB = 16
NW = 2 * N_SUB
K_EDGE = 128
CHUNKS = 80
E_PAD = NW * K_EDGE * CHUNKS

ROWS_PER_SUB = N_PAD // N_SUB


def _make_edge_agg(d, stage_h=False, tc_tiling=None):
    mesh = plsc.VectorSubcoreMesh(core_axis_name="c", subcore_axis_name="s")

    scratch = [
        pltpu.VMEM_SHARED((N_PAD, d), jnp.float32),
        pltpu.VMEM((CHUNKS, K_EDGE), jnp.int32),
        pltpu.VMEM((2, 2, K_EDGE), jnp.int32),
        pltpu.VMEM((2, K_EDGE, d), jnp.float32),
        pltpu.SemaphoreType.DMA((2,)),
        pltpu.SemaphoreType.DMA((2,)),
    ]
    if stage_h:
        scratch.insert(1, pltpu.VMEM_SHARED((N_PAD, d), jnp.float32))

    @functools.partial(
        pl.kernel,
        out_type=jax.ShapeDtypeStruct((2, N_PAD, d), jnp.float32),
        mesh=mesh,
        scratch_types=scratch,
        compiler_params=pltpu.CompilerParams(
            needs_layout_passes=False, use_tc_tiling_on_sc=tc_tiling
        ),
    )
    def agg(h_hbm, src_hbm, dw_hbm, out_hbm, *s):
        if stage_h:
            acc, h_sp, isrc, mbuf, rows, gsems, msems = s
        else:
            acc, isrc, mbuf, rows, gsems, msems = s
        h_src = h_sp if stage_h else h_hbm
        cid = lax.axis_index("c")
        sid = lax.axis_index("s")
        wid = cid * N_SUB + sid

        pltpu.sync_copy(src_hbm.at[wid], isrc)
        pltpu.sync_copy(dw_hbm.at[wid, 0], mbuf.at[0])
        if stage_h:
            pltpu.sync_copy(
                h_hbm.at[pl.ds(sid * ROWS_PER_SUB, ROWS_PER_SUB)],
                h_sp.at[pl.ds(sid * ROWS_PER_SUB, ROWS_PER_SUB)],
            )

        zero16 = jnp.zeros((16,), jnp.float32)

        def zrow(i, carry):
            for j in range(d // 16):
                rows[0, i, pl.ds(j * 16, 16)] = zero16
            return carry

        lax.fori_loop(0, K_EDGE, zrow, 0)
        for t in range(ROWS_PER_SUB // K_EDGE):
            pltpu.sync_copy(
                rows.at[0],
                acc.at[pl.ds(sid * ROWS_PER_SUB + t * K_EDGE, K_EDGE)],
            )
        plsc.subcore_barrier()

        pltpu.async_copy(h_src.at[isrc.at[0]], rows.at[0], gsems.at[0])

        @pl.loop(0, CHUNKS, step=2)
        def chunk2(c0):
            for b in range(2):
                c = c0 + b
                nxt = c + 1

                @pl.when(nxt < CHUNKS)
                def _():
                    pltpu.async_copy(
                        h_src.at[isrc.at[nxt]], rows.at[1 - b], gsems.at[1 - b]
                    )
                    pltpu.async_copy(
                        dw_hbm.at[wid, nxt], mbuf.at[1 - b], msems.at[1 - b]
                    )

                pltpu.make_async_copy(
                    h_src.at[isrc.at[c]], rows.at[b], gsems.at[b]
                ).wait()

                @pl.when(c > 0)
                def _():
                    pltpu.make_async_copy(
                        dw_hbm.at[wid, c], mbuf.at[b], msems.at[b]
                    ).wait()

                def grp(g, gc):
                    wv = plsc.bitcast(mbuf[b, 1, pl.ds(g * 16, 16)], jnp.float32)
                    for ii in range(16):
                        wb = wv.at[jnp.full((16,), ii, jnp.int32)].get(
                            mode="promise_in_bounds"
                        )
                        for j in range(d // 16):
                            rows[b, g * 16 + ii, pl.ds(j * 16, 16)] = (
                                rows[b, g * 16 + ii, pl.ds(j * 16, 16)] * wb
                            )
                    return gc

                lax.fori_loop(0, K_EDGE // 16, grp, 0)
                pltpu.sync_copy(rows.at[b], acc.at[mbuf.at[b, 0]], add=True)

        plsc.subcore_barrier()
        pltpu.sync_copy(
            acc.at[pl.ds(sid * ROWS_PER_SUB, ROWS_PER_SUB)],
            out_hbm.at[cid, pl.ds(sid * ROWS_PER_SUB, ROWS_PER_SUB)],
        )

    return agg


_edge_agg_h = _make_edge_agg(HIDDEN, stage_h=False, tc_tiling=None)
_edge_agg_c = _make_edge_agg(C_PAD, stage_h=True, tc_tiling=False)

_BM = 1024


def _mm1(x, w):
    def body(x_ref, w_ref, o_ref):
        o_ref[...] = jnp.dot(x_ref[...], w_ref[...], preferred_element_type=jnp.float32)

    return pl.pallas_call(
        body,
        grid=(N_PAD // _BM,),
        in_specs=[
            pl.BlockSpec((_BM, D_FEAT), lambda i: (i, 0)),
            pl.BlockSpec((D_FEAT, HIDDEN), lambda i: (0, 0)),
        ],
        out_specs=pl.BlockSpec((_BM, HIDDEN), lambda i: (i, 0)),
        out_shape=jax.ShapeDtypeStruct((N_PAD, HIDDEN), jnp.float32),
    )(x, w)


def _relu_mm2(p, b1, w2p):
    def body(p_ref, b_ref, w_ref, o_ref):
        h = jnp.maximum(p_ref[0] + p_ref[1] + b_ref[...], 0.0)
        o_ref[...] = jnp.dot(h, w_ref[...], preferred_element_type=jnp.float32)

    return pl.pallas_call(
        body,
        grid=(N_PAD // _BM,),
        in_specs=[
            pl.BlockSpec((2, _BM, HIDDEN), lambda i: (0, i, 0)),
            pl.BlockSpec((1, HIDDEN), lambda i: (0, 0)),
            pl.BlockSpec((HIDDEN, C_PAD), lambda i: (0, 0)),
        ],
        out_specs=pl.BlockSpec((_BM, C_PAD), lambda i: (i, 0)),
        out_shape=jax.ShapeDtypeStruct((N_PAD, C_PAD), jnp.float32),
    )(p, b1.reshape(1, HIDDEN), w2p)


def _log_softmax(q, b2):
    bm = 1000

    def body(q_ref, b_ref, o_ref):
        s = q_ref[0] + q_ref[1]
        logits = s[:, :N_CLASSES] + b_ref[...]
        m = jnp.max(logits, axis=1, keepdims=True)
        lse = jnp.log(jnp.sum(jnp.exp(logits - m), axis=1, keepdims=True)) + m
        o_ref[...] = logits - lse

    return pl.pallas_call(
        body,
        grid=(N_NODES // bm,),
        in_specs=[
            pl.BlockSpec((2, bm, C_PAD), lambda i: (0, i, 0)),
            pl.BlockSpec((1, N_CLASSES), lambda i: (0, 0)),
        ],
        out_specs=pl.BlockSpec((bm, N_CLASSES), lambda i: (i, 0)),
        out_shape=jax.ShapeDtypeStruct((N_NODES, N_CLASSES), jnp.float32),
    )(q, b2.reshape(1, N_CLASSES))


def kernel(x, edge_index, edge_weight, W1, b1, W2, b2):
    src = edge_index[0].astype(jnp.int32)
    dst = edge_index[1].astype(jnp.int32)
    pad = E_PAD - src.shape[0]
    shp = (NW, CHUNKS, K_EDGE)
    src_p = jnp.concatenate([src, jnp.zeros((pad,), jnp.int32)]).reshape(shp)
    dst_p = jnp.concatenate([dst, jnp.zeros((pad,), jnp.int32)]).reshape(shp)
    ew_bits = jax.lax.bitcast_convert_type(
        jnp.concatenate(
            [edge_weight.astype(jnp.float32), jnp.zeros((pad,), jnp.float32)]
        ),
        jnp.int32,
    ).reshape(shp)
    dw_p = jnp.stack([dst_p, ew_bits], axis=2)

    x_p = jnp.pad(x, ((0, N_PAD - N_NODES), (0, 0)))
    h1 = _mm1(x_p, W1)
    p1 = _edge_agg_h(h1, src_p, dw_p)
    w2p = jnp.pad(W2, ((0, 0), (0, C_PAD - N_CLASSES)))
    h2 = _relu_mm2(p1, b1, w2p)
    p2 = _edge_agg_c(h2, src_p, dw_p)
    return _log_softmax(p2, b2)


# speedup vs baseline: 1.2225x; 1.2225x over previous
import functools

import jax
import jax.numpy as jnp
from jax import lax
from jax.experimental import pallas as pl
from jax.experimental.pallas import tpu as pltpu
import jax.experimental.pallas.tpu_sc as plsc

N_NODES = 10000
N_PAD = 10240
D_FEAT = 128
HIDDEN = 128
N_CLASSES = 40
C_PAD = 48

N_SUB = 16
NW = 2 * N_SUB
K_EDGE = 128
CHUNKS = 80
E_PAD = NW * K_EDGE * CHUNKS

ROWS_PER_SUB = N_PAD // N_SUB


def _make_edge_agg(d, stage_h=False, tc_tiling=None):
    mesh = plsc.VectorSubcoreMesh(core_axis_name="c", subcore_axis_name="s")

    scratch = [
        pltpu.VMEM_SHARED((N_PAD, d), jnp.float32),
        pltpu.VMEM((CHUNKS, K_EDGE), jnp.int32),
        pltpu.VMEM((2, 2, K_EDGE), jnp.int32),
        pltpu.VMEM((2, K_EDGE, d), jnp.float32),
        pltpu.SemaphoreType.DMA((2,)),
        pltpu.SemaphoreType.DMA((2,)),
    ]
    if stage_h:
        scratch.insert(1, pltpu.VMEM_SHARED((N_PAD, d), jnp.float32))

    @functools.partial(
        pl.kernel,
        out_type=jax.ShapeDtypeStruct((2, N_PAD, d), jnp.float32),
        mesh=mesh,
        scratch_types=scratch,
        compiler_params=pltpu.CompilerParams(
            needs_layout_passes=False, use_tc_tiling_on_sc=tc_tiling
        ),
    )
    def agg(h_hbm, src_hbm, dw_hbm, out_hbm, *s):
        if stage_h:
            acc, h_sp, isrc, mbuf, rows, gsems, msems = s
        else:
            acc, isrc, mbuf, rows, gsems, msems = s
        h_src = h_sp if stage_h else h_hbm
        cid = lax.axis_index("c")
        sid = lax.axis_index("s")
        wid = cid * N_SUB + sid

        pltpu.sync_copy(src_hbm.at[wid], isrc)
        pltpu.sync_copy(dw_hbm.at[wid, 0], mbuf.at[0])
        if stage_h:
            pltpu.sync_copy(
                h_hbm.at[pl.ds(sid * ROWS_PER_SUB, ROWS_PER_SUB)],
                h_sp.at[pl.ds(sid * ROWS_PER_SUB, ROWS_PER_SUB)],
            )

        zero16 = jnp.zeros((16,), jnp.float32)

        def zrow(i, carry):
            for j in range(d // 16):
                rows[0, i, pl.ds(j * 16, 16)] = zero16
            return carry

        lax.fori_loop(0, K_EDGE, zrow, 0)
        for t in range(ROWS_PER_SUB // K_EDGE):
            pltpu.sync_copy(
                rows.at[0],
                acc.at[pl.ds(sid * ROWS_PER_SUB + t * K_EDGE, K_EDGE)],
            )
        plsc.subcore_barrier()

        pltpu.async_copy(h_src.at[isrc.at[0]], rows.at[0], gsems.at[0])

        @pl.loop(0, CHUNKS, step=2)
        def chunk2(c0):
            for b in range(2):
                c = c0 + b
                nxt = c + 1

                @pl.when(nxt < CHUNKS)
                def _():
                    pltpu.async_copy(
                        h_src.at[isrc.at[nxt]], rows.at[1 - b], gsems.at[1 - b]
                    )
                    pltpu.async_copy(
                        dw_hbm.at[wid, nxt], mbuf.at[1 - b], msems.at[1 - b]
                    )

                pltpu.make_async_copy(
                    h_src.at[isrc.at[c]], rows.at[b], gsems.at[b]
                ).wait()

                @pl.when(c > 0)
                def _():
                    pltpu.make_async_copy(
                        dw_hbm.at[wid, c], mbuf.at[b], msems.at[b]
                    ).wait()

                def grp(g, gc):
                    wv = plsc.bitcast(mbuf[b, 1, pl.ds(g * 16, 16)], jnp.float32)
                    for ii in range(16):
                        wb = wv.at[jnp.full((16,), ii, jnp.int32)].get(
                            mode="promise_in_bounds"
                        )
                        for j in range(d // 16):
                            rows[b, g * 16 + ii, pl.ds(j * 16, 16)] = (
                                rows[b, g * 16 + ii, pl.ds(j * 16, 16)] * wb
                            )
                    return gc

                lax.fori_loop(0, K_EDGE // 16, grp, 0)
                pltpu.sync_copy(rows.at[b], acc.at[mbuf.at[b, 0]], add=True)

        plsc.subcore_barrier()
        pltpu.sync_copy(
            acc.at[pl.ds(sid * ROWS_PER_SUB, ROWS_PER_SUB)],
            out_hbm.at[cid, pl.ds(sid * ROWS_PER_SUB, ROWS_PER_SUB)],
        )

    return agg


D_HALF = HIDDEN // 2
CHUNKS_SPLIT = E_PAD // (N_SUB * K_EDGE)


def _make_edge_agg_split(dch, chunks):
    mesh = plsc.VectorSubcoreMesh(core_axis_name="c", subcore_axis_name="s")

    @functools.partial(
        pl.kernel,
        out_type=jax.ShapeDtypeStruct((2, N_PAD, dch), jnp.float32),
        mesh=mesh,
        scratch_types=[
            pltpu.VMEM_SHARED((N_PAD, dch), jnp.float32),
            pltpu.VMEM_SHARED((N_PAD, dch), jnp.float32),
            pltpu.VMEM((chunks, K_EDGE), jnp.int32),
            pltpu.VMEM((2, 2, K_EDGE), jnp.int32),
            pltpu.VMEM((2, K_EDGE, dch), jnp.float32),
            pltpu.SemaphoreType.DMA((2,)),
            pltpu.SemaphoreType.DMA((2,)),
        ],
        compiler_params=pltpu.CompilerParams(
            needs_layout_passes=False, use_tc_tiling_on_sc=False
        ),
    )
    def agg(h_hbm, src_hbm, dw_hbm, out_hbm, acc, h_sp, isrc, mbuf, rows, gsems, msems):
        cid = lax.axis_index("c")
        sid = lax.axis_index("s")

        pltpu.sync_copy(src_hbm.at[sid], isrc)
        pltpu.sync_copy(dw_hbm.at[sid, 0], mbuf.at[0])
        pltpu.sync_copy(
            h_hbm.at[cid, pl.ds(sid * ROWS_PER_SUB, ROWS_PER_SUB)],
            h_sp.at[pl.ds(sid * ROWS_PER_SUB, ROWS_PER_SUB)],
        )

        zero16 = jnp.zeros((16,), jnp.float32)

        def zrow(i, carry):
            for j in range(dch // 16):
                rows[0, i, pl.ds(j * 16, 16)] = zero16
            return carry

        lax.fori_loop(0, K_EDGE, zrow, 0)
        for t in range(ROWS_PER_SUB // K_EDGE):
            pltpu.sync_copy(
                rows.at[0],
                acc.at[pl.ds(sid * ROWS_PER_SUB + t * K_EDGE, K_EDGE)],
            )
        plsc.subcore_barrier()

        pltpu.async_copy(h_sp.at[isrc.at[0]], rows.at[0], gsems.at[0])

        @pl.loop(0, chunks, step=2)
        def chunk2(c0):
            for b in range(2):
                c = c0 + b
                nxt = c + 1

                @pl.when(nxt < chunks)
                def _():
                    pltpu.async_copy(
                        h_sp.at[isrc.at[nxt]], rows.at[1 - b], gsems.at[1 - b]
                    )
                    pltpu.async_copy(
                        dw_hbm.at[sid, nxt], mbuf.at[1 - b], msems.at[1 - b]
                    )

                pltpu.make_async_copy(
                    h_sp.at[isrc.at[c]], rows.at[b], gsems.at[b]
                ).wait()

                @pl.when(c > 0)
                def _():
                    pltpu.make_async_copy(
                        dw_hbm.at[sid, c], mbuf.at[b], msems.at[b]
                    ).wait()

                def grp(g, gc):
                    wv = plsc.bitcast(mbuf[b, 1, pl.ds(g * 16, 16)], jnp.float32)
                    for ii in range(16):
                        wb = wv.at[jnp.full((16,), ii, jnp.int32)].get(
                            mode="promise_in_bounds"
                        )
                        for j in range(dch // 16):
                            rows[b, g * 16 + ii, pl.ds(j * 16, 16)] = (
                                rows[b, g * 16 + ii, pl.ds(j * 16, 16)] * wb
                            )
                    return gc

                lax.fori_loop(0, K_EDGE // 16, grp, 0)
                pltpu.sync_copy(rows.at[b], acc.at[mbuf.at[b, 0]], add=True)

        plsc.subcore_barrier()
        pltpu.sync_copy(
            acc.at[pl.ds(sid * ROWS_PER_SUB, ROWS_PER_SUB)],
            out_hbm.at[cid, pl.ds(sid * ROWS_PER_SUB, ROWS_PER_SUB)],
        )

    return agg


_edge_agg_h = _make_edge_agg_split(D_HALF, CHUNKS_SPLIT)
_edge_agg_c = _make_edge_agg(C_PAD, stage_h=True, tc_tiling=False)

_BM = 1024


def _mm1(x, w):
    def body(x_ref, w_ref, o_ref):
        o_ref[0] = jnp.dot(x_ref[...], w_ref[0], preferred_element_type=jnp.float32)

    return pl.pallas_call(
        body,
        grid=(2, N_PAD // _BM),
        in_specs=[
            pl.BlockSpec((_BM, D_FEAT), lambda c, i: (i, 0)),
            pl.BlockSpec((1, D_FEAT, D_HALF), lambda c, i: (c, 0, 0)),
        ],
        out_specs=pl.BlockSpec((1, _BM, D_HALF), lambda c, i: (c, i, 0)),
        out_shape=jax.ShapeDtypeStruct((2, N_PAD, D_HALF), jnp.float32),
    )(x, w)


def _relu_mm2(p, b1, w2p):
    def body(p_ref, b_ref, w_ref, o_ref):
        agg1 = jnp.concatenate([p_ref[0], p_ref[1]], axis=-1)
        h = jnp.maximum(agg1 + b_ref[...], 0.0)
        o_ref[...] = jnp.dot(h, w_ref[...], preferred_element_type=jnp.float32)

    return pl.pallas_call(
        body,
        grid=(N_PAD // _BM,),
        in_specs=[
            pl.BlockSpec((2, _BM, D_HALF), lambda i: (0, i, 0)),
            pl.BlockSpec((1, HIDDEN), lambda i: (0, 0)),
            pl.BlockSpec((HIDDEN, C_PAD), lambda i: (0, 0)),
        ],
        out_specs=pl.BlockSpec((_BM, C_PAD), lambda i: (i, 0)),
        out_shape=jax.ShapeDtypeStruct((N_PAD, C_PAD), jnp.float32),
    )(p, b1.reshape(1, HIDDEN), w2p)


def _log_softmax(q, b2):
    bm = 1000

    def body(q_ref, b_ref, o_ref):
        s = q_ref[0] + q_ref[1]
        logits = s[:, :N_CLASSES] + b_ref[...]
        m = jnp.max(logits, axis=1, keepdims=True)
        lse = jnp.log(jnp.sum(jnp.exp(logits - m), axis=1, keepdims=True)) + m
        o_ref[...] = logits - lse

    return pl.pallas_call(
        body,
        grid=(N_NODES // bm,),
        in_specs=[
            pl.BlockSpec((2, bm, C_PAD), lambda i: (0, i, 0)),
            pl.BlockSpec((1, N_CLASSES), lambda i: (0, 0)),
        ],
        out_specs=pl.BlockSpec((bm, N_CLASSES), lambda i: (i, 0)),
        out_shape=jax.ShapeDtypeStruct((N_NODES, N_CLASSES), jnp.float32),
    )(q, b2.reshape(1, N_CLASSES))


def kernel(x, edge_index, edge_weight, W1, b1, W2, b2):
    src = edge_index[0].astype(jnp.int32)
    dst = edge_index[1].astype(jnp.int32)
    pad = E_PAD - src.shape[0]
    shp = (NW, CHUNKS, K_EDGE)
    src_p = jnp.concatenate([src, jnp.zeros((pad,), jnp.int32)]).reshape(shp)
    dst_p = jnp.concatenate([dst, jnp.zeros((pad,), jnp.int32)]).reshape(shp)
    ew_bits = jax.lax.bitcast_convert_type(
        jnp.concatenate(
            [edge_weight.astype(jnp.float32), jnp.zeros((pad,), jnp.float32)]
        ),
        jnp.int32,
    ).reshape(shp)
    dw_p = jnp.stack([dst_p, ew_bits], axis=2)
    shp_s = (N_SUB, CHUNKS_SPLIT, K_EDGE)
    src_s = src_p.reshape(shp_s)
    dw_s = jnp.stack(
        [dst_p.reshape(shp_s), ew_bits.reshape(shp_s)], axis=2
    )

    x_p = jnp.pad(x, ((0, N_PAD - N_NODES), (0, 0)))
    w1_s = W1.reshape(D_FEAT, 2, D_HALF).transpose(1, 0, 2)
    h1 = _mm1(x_p, w1_s)
    p1 = _edge_agg_h(h1, src_s, dw_s)
    w2p = jnp.pad(W2, ((0, 0), (0, C_PAD - N_CLASSES)))
    h2 = _relu_mm2(p1, b1, w2p)
    p2 = _edge_agg_c(h2, src_p, dw_p)
    return _log_softmax(p2, b2)


# speedup vs baseline: 1.9570x; 1.6009x over previous
import functools

import jax
import jax.numpy as jnp
from jax import lax
from jax.experimental import pallas as pl
from jax.experimental.pallas import tpu as pltpu
import jax.experimental.pallas.tpu_sc as plsc

N_NODES = 10000
N_PAD = 10240
D_FEAT = 128
HIDDEN = 128
N_CLASSES = 40
C_PAD = 48

N_SUB = 16
NW = 2 * N_SUB
K_EDGE = 128
CHUNKS = 80
E_PAD = NW * K_EDGE * CHUNKS

ROWS_PER_SUB = N_PAD // N_SUB


def _make_edge_agg(d, stage_h=False, tc_tiling=None):
    mesh = plsc.VectorSubcoreMesh(core_axis_name="c", subcore_axis_name="s")

    scratch = [
        pltpu.VMEM_SHARED((N_PAD, d), jnp.float32),
        pltpu.VMEM((CHUNKS, K_EDGE), jnp.int32),
        pltpu.VMEM((2, 2, K_EDGE), jnp.int32),
        pltpu.VMEM((2, K_EDGE, d), jnp.float32),
        pltpu.SemaphoreType.DMA((2,)),
        pltpu.SemaphoreType.DMA((2,)),
        pltpu.SemaphoreType.DMA((2,)),
    ]
    if stage_h:
        scratch.insert(1, pltpu.VMEM_SHARED((N_PAD, d), jnp.float32))

    @functools.partial(
        pl.kernel,
        out_type=jax.ShapeDtypeStruct((2, N_PAD, d), jnp.float32),
        mesh=mesh,
        scratch_types=scratch,
        compiler_params=pltpu.CompilerParams(
            needs_layout_passes=False, use_tc_tiling_on_sc=tc_tiling
        ),
    )
    def agg(h_hbm, src_hbm, dw_hbm, out_hbm, *s):
        if stage_h:
            acc, h_sp, isrc, mbuf, rows, gsems, msems, ssems = s
        else:
            acc, isrc, mbuf, rows, gsems, msems, ssems = s
        h_src = h_sp if stage_h else h_hbm
        cid = lax.axis_index("c")
        sid = lax.axis_index("s")
        wid = cid * N_SUB + sid

        pltpu.sync_copy(src_hbm.at[wid], isrc)
        pltpu.sync_copy(dw_hbm.at[wid, 0], mbuf.at[0])
        if stage_h:
            pltpu.sync_copy(
                h_hbm.at[pl.ds(sid * ROWS_PER_SUB, ROWS_PER_SUB)],
                h_sp.at[pl.ds(sid * ROWS_PER_SUB, ROWS_PER_SUB)],
            )

        zero16 = jnp.zeros((16,), jnp.float32)

        def zrow(i, carry):
            for j in range(d // 16):
                rows[0, i, pl.ds(j * 16, 16)] = zero16
            return carry

        lax.fori_loop(0, K_EDGE, zrow, 0)
        for t in range(ROWS_PER_SUB // K_EDGE):
            pltpu.sync_copy(
                rows.at[0],
                acc.at[pl.ds(sid * ROWS_PER_SUB + t * K_EDGE, K_EDGE)],
            )
        plsc.subcore_barrier()

        pltpu.async_copy(h_src.at[isrc.at[0]], rows.at[0], gsems.at[0])

        @pl.loop(0, CHUNKS, step=2)
        def chunk2(c0):
            for b in range(2):
                c = c0 + b
                nxt = c + 1

                @pl.when(c > 0)
                def _():
                    pltpu.make_async_copy(
                        rows.at[1 - b], acc.at[mbuf.at[1 - b, 0]], ssems.at[1 - b]
                    ).wait()

                @pl.when(nxt < CHUNKS)
                def _():
                    pltpu.async_copy(
                        h_src.at[isrc.at[nxt]], rows.at[1 - b], gsems.at[1 - b]
                    )
                    pltpu.async_copy(
                        dw_hbm.at[wid, nxt], mbuf.at[1 - b], msems.at[1 - b]
                    )

                pltpu.make_async_copy(
                    h_src.at[isrc.at[c]], rows.at[b], gsems.at[b]
                ).wait()

                @pl.when(c > 0)
                def _():
                    pltpu.make_async_copy(
                        dw_hbm.at[wid, c], mbuf.at[b], msems.at[b]
                    ).wait()

                for g in range(K_EDGE // 16):
                    wv = plsc.bitcast(mbuf[b, 1, pl.ds(g * 16, 16)], jnp.float32)
                    for ii in range(16):
                        wb = wv.at[jnp.full((16,), ii, jnp.int32)].get(
                            mode="promise_in_bounds"
                        )
                        for j in range(d // 16):
                            rows[b, g * 16 + ii, pl.ds(j * 16, 16)] = (
                                rows[b, g * 16 + ii, pl.ds(j * 16, 16)] * wb
                            )

                pltpu.async_copy(rows.at[b], acc.at[mbuf.at[b, 0]], ssems.at[b], add=True)

        pltpu.make_async_copy(
            rows.at[1], acc.at[mbuf.at[1, 0]], ssems.at[1]
        ).wait()
        plsc.subcore_barrier()
        pltpu.sync_copy(
            acc.at[pl.ds(sid * ROWS_PER_SUB, ROWS_PER_SUB)],
            out_hbm.at[cid, pl.ds(sid * ROWS_PER_SUB, ROWS_PER_SUB)],
        )

    return agg


D_HALF = HIDDEN // 2
CHUNKS_SPLIT = E_PAD // (N_SUB * K_EDGE)


def _make_edge_agg_split(dch, chunks):
    mesh = plsc.VectorSubcoreMesh(core_axis_name="c", subcore_axis_name="s")

    @functools.partial(
        pl.kernel,
        out_type=jax.ShapeDtypeStruct((2, N_PAD, dch), jnp.float32),
        mesh=mesh,
        scratch_types=[
            pltpu.VMEM_SHARED((N_PAD, dch), jnp.float32),
            pltpu.VMEM_SHARED((N_PAD, dch), jnp.float32),
            pltpu.VMEM((chunks, K_EDGE), jnp.int32),
            pltpu.VMEM((2, 2, K_EDGE), jnp.int32),
            pltpu.VMEM((2, K_EDGE, dch), jnp.float32),
            pltpu.SemaphoreType.DMA((2,)),
            pltpu.SemaphoreType.DMA((2,)),
            pltpu.SemaphoreType.DMA((2,)),
        ],
        compiler_params=pltpu.CompilerParams(
            needs_layout_passes=False, use_tc_tiling_on_sc=False
        ),
    )
    def agg(h_hbm, src_hbm, dw_hbm, out_hbm, acc, h_sp, isrc, mbuf, rows, gsems, msems, ssems):
        cid = lax.axis_index("c")
        sid = lax.axis_index("s")

        pltpu.sync_copy(src_hbm.at[sid], isrc)
        pltpu.sync_copy(dw_hbm.at[sid, 0], mbuf.at[0])
        pltpu.sync_copy(
            h_hbm.at[cid, pl.ds(sid * ROWS_PER_SUB, ROWS_PER_SUB)],
            h_sp.at[pl.ds(sid * ROWS_PER_SUB, ROWS_PER_SUB)],
        )

        zero16 = jnp.zeros((16,), jnp.float32)

        def zrow(i, carry):
            for j in range(dch // 16):
                rows[0, i, pl.ds(j * 16, 16)] = zero16
            return carry

        lax.fori_loop(0, K_EDGE, zrow, 0)
        for t in range(ROWS_PER_SUB // K_EDGE):
            pltpu.sync_copy(
                rows.at[0],
                acc.at[pl.ds(sid * ROWS_PER_SUB + t * K_EDGE, K_EDGE)],
            )
        plsc.subcore_barrier()

        pltpu.async_copy(h_sp.at[isrc.at[0]], rows.at[0], gsems.at[0])

        @pl.loop(0, chunks, step=2)
        def chunk2(c0):
            for b in range(2):
                c = c0 + b
                nxt = c + 1

                @pl.when(c > 0)
                def _():
                    pltpu.make_async_copy(
                        rows.at[1 - b], acc.at[mbuf.at[1 - b, 0]], ssems.at[1 - b]
                    ).wait()

                @pl.when(nxt < chunks)
                def _():
                    pltpu.async_copy(
                        h_sp.at[isrc.at[nxt]], rows.at[1 - b], gsems.at[1 - b]
                    )
                    pltpu.async_copy(
                        dw_hbm.at[sid, nxt], mbuf.at[1 - b], msems.at[1 - b]
                    )

                pltpu.make_async_copy(
                    h_sp.at[isrc.at[c]], rows.at[b], gsems.at[b]
                ).wait()

                @pl.when(c > 0)
                def _():
                    pltpu.make_async_copy(
                        dw_hbm.at[sid, c], mbuf.at[b], msems.at[b]
                    ).wait()

                for g in range(K_EDGE // 16):
                    wv = plsc.bitcast(mbuf[b, 1, pl.ds(g * 16, 16)], jnp.float32)
                    for ii in range(16):
                        wb = wv.at[jnp.full((16,), ii, jnp.int32)].get(
                            mode="promise_in_bounds"
                        )
                        for j in range(dch // 16):
                            rows[b, g * 16 + ii, pl.ds(j * 16, 16)] = (
                                rows[b, g * 16 + ii, pl.ds(j * 16, 16)] * wb
                            )

                pltpu.async_copy(rows.at[b], acc.at[mbuf.at[b, 0]], ssems.at[b], add=True)

        pltpu.make_async_copy(
            rows.at[1], acc.at[mbuf.at[1, 0]], ssems.at[1]
        ).wait()
        plsc.subcore_barrier()
        pltpu.sync_copy(
            acc.at[pl.ds(sid * ROWS_PER_SUB, ROWS_PER_SUB)],
            out_hbm.at[cid, pl.ds(sid * ROWS_PER_SUB, ROWS_PER_SUB)],
        )

    return agg


_edge_agg_h = _make_edge_agg_split(D_HALF, CHUNKS_SPLIT)
_edge_agg_c = _make_edge_agg(C_PAD, stage_h=True, tc_tiling=False)

_BM = 1024


def _mm1(x, w):
    def body(x_ref, w_ref, o_ref):
        o_ref[0] = jnp.dot(x_ref[...], w_ref[0], preferred_element_type=jnp.float32)

    return pl.pallas_call(
        body,
        grid=(2, N_PAD // _BM),
        in_specs=[
            pl.BlockSpec((_BM, D_FEAT), lambda c, i: (i, 0)),
            pl.BlockSpec((1, D_FEAT, D_HALF), lambda c, i: (c, 0, 0)),
        ],
        out_specs=pl.BlockSpec((1, _BM, D_HALF), lambda c, i: (c, i, 0)),
        out_shape=jax.ShapeDtypeStruct((2, N_PAD, D_HALF), jnp.float32),
    )(x, w)


def _relu_mm2(p, b1, w2p):
    def body(p_ref, b_ref, w_ref, o_ref):
        agg1 = jnp.concatenate([p_ref[0], p_ref[1]], axis=-1)
        h = jnp.maximum(agg1 + b_ref[...], 0.0)
        o_ref[...] = jnp.dot(h, w_ref[...], preferred_element_type=jnp.float32)

    return pl.pallas_call(
        body,
        grid=(N_PAD // _BM,),
        in_specs=[
            pl.BlockSpec((2, _BM, D_HALF), lambda i: (0, i, 0)),
            pl.BlockSpec((1, HIDDEN), lambda i: (0, 0)),
            pl.BlockSpec((HIDDEN, C_PAD), lambda i: (0, 0)),
        ],
        out_specs=pl.BlockSpec((_BM, C_PAD), lambda i: (i, 0)),
        out_shape=jax.ShapeDtypeStruct((N_PAD, C_PAD), jnp.float32),
    )(p, b1.reshape(1, HIDDEN), w2p)


def _log_softmax(q, b2):
    bm = 1000

    def body(q_ref, b_ref, o_ref):
        s = q_ref[0] + q_ref[1]
        logits = s[:, :N_CLASSES] + b_ref[...]
        m = jnp.max(logits, axis=1, keepdims=True)
        lse = jnp.log(jnp.sum(jnp.exp(logits - m), axis=1, keepdims=True)) + m
        o_ref[...] = logits - lse

    return pl.pallas_call(
        body,
        grid=(N_NODES // bm,),
        in_specs=[
            pl.BlockSpec((2, bm, C_PAD), lambda i: (0, i, 0)),
            pl.BlockSpec((1, N_CLASSES), lambda i: (0, 0)),
        ],
        out_specs=pl.BlockSpec((bm, N_CLASSES), lambda i: (i, 0)),
        out_shape=jax.ShapeDtypeStruct((N_NODES, N_CLASSES), jnp.float32),
    )(q, b2.reshape(1, N_CLASSES))


def kernel(x, edge_index, edge_weight, W1, b1, W2, b2):
    src = edge_index[0].astype(jnp.int32)
    dst = edge_index[1].astype(jnp.int32)
    pad = E_PAD - src.shape[0]
    shp = (NW, CHUNKS, K_EDGE)
    src_p = jnp.concatenate([src, jnp.zeros((pad,), jnp.int32)]).reshape(shp)
    dst_p = jnp.concatenate([dst, jnp.zeros((pad,), jnp.int32)]).reshape(shp)
    ew_bits = jax.lax.bitcast_convert_type(
        jnp.concatenate(
            [edge_weight.astype(jnp.float32), jnp.zeros((pad,), jnp.float32)]
        ),
        jnp.int32,
    ).reshape(shp)
    dw_p = jnp.stack([dst_p, ew_bits], axis=2)
    shp_s = (N_SUB, CHUNKS_SPLIT, K_EDGE)
    src_s = src_p.reshape(shp_s)
    dw_s = jnp.stack(
        [dst_p.reshape(shp_s), ew_bits.reshape(shp_s)], axis=2
    )

    x_p = jnp.pad(x, ((0, N_PAD - N_NODES), (0, 0)))
    w1_s = W1.reshape(D_FEAT, 2, D_HALF).transpose(1, 0, 2)
    h1 = _mm1(x_p, w1_s)
    p1 = _edge_agg_h(h1, src_s, dw_s)
    w2p = jnp.pad(W2, ((0, 0), (0, C_PAD - N_CLASSES)))
    h2 = _relu_mm2(p1, b1, w2p)
    p2 = _edge_agg_c(h2, src_p, dw_p)
    return _log_softmax(p2, b2)
